# baseline (speedup 1.0000x reference)
import functools

import jax
import jax.numpy as jnp
from jax import lax
from jax.experimental import pallas as pl
from jax.experimental.pallas import tpu as pltpu
from jax.experimental.pallas import tpu_sc as plsc

B, S, D, DFF = 4, 4096, 2048, 8192
K = S // 8
BK = B * K

NC, NS, LANES = 2, 16, 16
NW = NC * NS
ROWS_PER_W = BK // NW
CHUNK = 32
NCHUNK = ROWS_PER_W // CHUNK

_SC_MESH = dict(core_axis_name="c", subcore_axis_name="s", num_cores=NC,
                num_subcores=NS)


def _thresh_body(sig_ref, thr_ref):
    keys = lax.bitcast_convert_type(sig_ref[...], jnp.int32)

    def body(_, lohi):
        lo, hi = lohi
        mid = lo + ((hi - lo + 1) >> 1)
        cnt = jnp.sum((keys >= mid).astype(jnp.int32), axis=1, keepdims=True)
        p = cnt >= K
        return jnp.where(p, mid, lo), jnp.where(p, hi, mid - 1)

    lo0 = jnp.zeros((B, 1), jnp.int32)
    hi0 = jnp.full((B, 1), 0x3F800000, jnp.int32)
    lo, _ = lax.fori_loop(0, 31, body, (lo0, hi0))
    thr_ref[...] = jnp.broadcast_to(
        lax.bitcast_convert_type(lo, jnp.float32), (B, LANES))


def _thresholds(sig):
    return pl.pallas_call(
        _thresh_body,
        out_shape=jax.ShapeDtypeStruct((B, LANES), jnp.float32),
    )(sig)


CPROWS = B * S // NW


def _copy_body(x_hbm, o_hbm):
    wid = lax.axis_index("s") * NC + lax.axis_index("c")
    base = wid * CPROWS
    pltpu.sync_copy(x_hbm.at[pl.ds(base, CPROWS)],
                    o_hbm.at[pl.ds(base, CPROWS)])


def _copy(x_flat):
    f = pl.kernel(
        _copy_body,
        out_type=jax.ShapeDtypeStruct((B * S, D), jnp.float32),
        mesh=plsc.VectorSubcoreMesh(**_SC_MESH),
        compiler_params=pltpu.CompilerParams(needs_layout_passes=False),
    )
    return f(x_flat)


def _emit_body(sig_hbm, thr_hbm, idx_hbm, w_hbm, sig_v, idx_v, wbuf_v, thr_v,
               cnt_v):
    wid = lax.axis_index("s") * NC + lax.axis_index("c")

    @pl.when(wid < B)
    def _():
        b = wid
        pltpu.sync_copy(sig_hbm.at[b], sig_v)
        pltpu.sync_copy(thr_hbm.at[b], thr_v)
        t = thr_v[...]
        ones = jnp.full((LANES,), 1, jnp.int32)
        zeros = jnp.full((LANES,), 0, jnp.int32)
        cnt_v[...] = zeros

        def emit(pred):
            def step(j, _):
                v = sig_v[pl.ds(j * LANES, LANES)]
                m = pred(v)
                mi = jnp.where(m, ones, zeros)
                cnt_vec = cnt_v[...]
                pos = cnt_vec + plsc.cumsum(mi) - mi
                gidx = b * S + j * LANES + lax.iota(jnp.int32, LANES)
                plsc.store_scatter(idx_v, [pos], gidx, mask=m)
                plsc.store_scatter(wbuf_v, [pos], v, mask=m)
                cnt_v[...] = cnt_vec + plsc.all_reduce_population_count(m)
                return 0
            return step

        lax.fori_loop(0, S // LANES, emit(lambda v: v > t), 0)
        lax.fori_loop(0, S // LANES, emit(lambda v: v == t), 0)

        pltpu.sync_copy(wbuf_v.at[pl.ds(0, K)], w_hbm.at[b])
        pltpu.sync_copy(idx_v.at[pl.ds(0, K)], idx_hbm.at[pl.ds(b * K, K)])


def _emit(sig, thr):
    f = pl.kernel(
        _emit_body,
        out_type=[jax.ShapeDtypeStruct((BK,), jnp.int32),
                  jax.ShapeDtypeStruct((B, K), jnp.float32)],
        mesh=plsc.VectorSubcoreMesh(**_SC_MESH),
        scratch_types=[pltpu.VMEM((S,), jnp.float32),
                       pltpu.VMEM((S + 2 * LANES,), jnp.int32),
                       pltpu.VMEM((S + 2 * LANES,), jnp.float32),
                       pltpu.VMEM((LANES,), jnp.float32),
                       pltpu.VMEM((LANES,), jnp.int32)],
        compiler_params=pltpu.CompilerParams(needs_layout_passes=False),
    )
    return f(sig, thr)


def _gather_body(x_hbm, idx_hbm, xs_hbm, idx_v, rows_v, sem):
    wid = lax.axis_index("s") * NC + lax.axis_index("c")
    base = wid * ROWS_PER_W
    for c in range(NCHUNK):
        pltpu.sync_copy(idx_hbm.at[pl.ds(base + c * CHUNK, CHUNK)],
                        idx_v.at[c])
        pltpu.async_copy(x_hbm.at[idx_v.at[c]], rows_v, sem).wait()
        pltpu.sync_copy(rows_v, xs_hbm.at[pl.ds(base + c * CHUNK, CHUNK)])


def _gather(x_flat, idx):
    f = pl.kernel(
        _gather_body,
        out_type=jax.ShapeDtypeStruct((BK, D), jnp.float32),
        mesh=plsc.VectorSubcoreMesh(**_SC_MESH),
        scratch_types=[pltpu.VMEM((NCHUNK, CHUNK), jnp.int32),
                       pltpu.VMEM((CHUNK, D), jnp.float32),
                       pltpu.SemaphoreType.DMA],
    )
    return f(x_flat, idx)


FBLK = 512
NF = DFF // FBLK


def _ffn_body(xs_ref, w_ref, w1_ref, w2_ref, y_ref):
    f = pl.program_id(0)

    @pl.when(f == 0)
    def _():
        y_ref[...] = xs_ref[...].astype(jnp.float32)

    w1b = w1_ref[...].astype(jnp.bfloat16)
    h = jnp.maximum(
        jnp.dot(xs_ref[...], w1b, preferred_element_type=jnp.float32), 0.0)
    hb = h.astype(jnp.bfloat16)
    w2b = w2_ref[...].astype(jnp.bfloat16)
    y_ref[...] += w_ref[...] * jnp.dot(hb, w2b,
                                       preferred_element_type=jnp.float32)


def _ffn(xs_bf, w16, W1, W2):
    return pl.pallas_call(
        _ffn_body,
        grid=(NF,),
        in_specs=[
            pl.BlockSpec((BK, D), lambda f: (0, 0)),
            pl.BlockSpec((BK, 1), lambda f: (0, 0)),
            pl.BlockSpec((D, FBLK), lambda f: (0, f)),
            pl.BlockSpec((FBLK, D), lambda f: (f, 0)),
        ],
        out_specs=pl.BlockSpec((BK, D), lambda f: (0, 0)),
        out_shape=jax.ShapeDtypeStruct((BK, D), jnp.float32),
        compiler_params=pltpu.CompilerParams(
            dimension_semantics=("arbitrary",)),
    )(xs_bf, w16, W1, W2)


def _scatter_body(y_hbm, idx_hbm, out_ref, idx_v, rows_v, sem):
    wid = lax.axis_index("s") * NC + lax.axis_index("c")
    base = wid * ROWS_PER_W
    for c in range(NCHUNK):
        pltpu.sync_copy(idx_hbm.at[pl.ds(base + c * CHUNK, CHUNK)],
                        idx_v.at[c])
        pltpu.sync_copy(y_hbm.at[pl.ds(base + c * CHUNK, CHUNK)], rows_v)
        pltpu.async_copy(rows_v, out_ref.at[idx_v.at[c]], sem).wait()


def _scatter(y, idx, out_ref):
    f = pl.kernel(
        _scatter_body,
        out_type=(),
        mesh=plsc.VectorSubcoreMesh(**_SC_MESH),
        scratch_types=[pltpu.VMEM((NCHUNK, CHUNK), jnp.int32),
                       pltpu.VMEM((CHUNK, D), jnp.float32),
                       pltpu.SemaphoreType.DMA],
    )
    f(y, idx, out_ref)


def kernel(x, gate_W, W1, W2):
    sig = jax.nn.sigmoid(
        jnp.einsum('bsd,do->bso', x.astype(jnp.float32), gate_W))[..., 0]

    x_flat = x.reshape(B * S, D)
    thr = _thresholds(sig)
    idx, w = _emit(sig, thr)
    xs = _gather(x_flat, idx)
    out_copy = _copy(x_flat)
    xs_bf = xs.astype(jnp.bfloat16)
    w16 = w.astype(jnp.float16).astype(jnp.float32).reshape(BK, 1)
    y = _ffn(xs_bf, w16, W1, W2)

    out_ref = jax.new_ref(out_copy)
    _scatter(y, idx, out_ref)
    return out_ref[...].reshape(B, S, D)


# speedup vs baseline: 11.8877x; 11.8877x over previous
import functools

import jax
import jax.numpy as jnp
from jax import lax
from jax.experimental import pallas as pl
from jax.experimental.pallas import tpu as pltpu
from jax.experimental.pallas import tpu_sc as plsc

B, S, D, DFF = 4, 4096, 2048, 8192
K = S // 8
BK = B * K

NC, NS, LANES = 2, 16, 16
NW = NC * NS
ROWS_PER_W = BK // NW
CHUNK = 32
NCHUNK = ROWS_PER_W // CHUNK

_SC_MESH = dict(core_axis_name="c", subcore_axis_name="s", num_cores=NC,
                num_subcores=NS)


def _thresh_body(sig_ref, thr_ref):
    keys = lax.bitcast_convert_type(sig_ref[...], jnp.int32)

    def body(_, lohi):
        lo, hi = lohi
        mid = lo + ((hi - lo + 1) >> 1)
        cnt = jnp.sum((keys >= mid).astype(jnp.int32), axis=1, keepdims=True)
        p = cnt >= K
        return jnp.where(p, mid, lo), jnp.where(p, hi, mid - 1)

    lo0 = jnp.zeros((B, 1), jnp.int32)
    hi0 = jnp.full((B, 1), 0x3F800000, jnp.int32)
    lo, _ = lax.fori_loop(0, 31, body, (lo0, hi0))
    thr_ref[...] = jnp.broadcast_to(
        lax.bitcast_convert_type(lo, jnp.float32), (B, LANES))


def _thresholds(sig):
    return pl.pallas_call(
        _thresh_body,
        out_shape=jax.ShapeDtypeStruct((B, LANES), jnp.float32),
    )(sig)


def _copy_body(x_ref, o_ref):
    o_ref[...] = x_ref[...]


def _copy(x):
    blk = 256
    return pl.pallas_call(
        _copy_body,
        grid=(S // blk,),
        in_specs=[pl.BlockSpec((B, blk, D), lambda i: (0, i, 0))],
        out_specs=pl.BlockSpec((B, blk, D), lambda i: (0, i, 0)),
        out_shape=jax.ShapeDtypeStruct((B, S, D), jnp.float32),
    )(x)


def _emit_body(sig_hbm, thr_hbm, idx_hbm, w_hbm, sig_v, idx_v, wbuf_v, thr_v,
               cnt_v):
    wid = lax.axis_index("s") * NC + lax.axis_index("c")

    @pl.when(wid < B)
    def _():
        b = wid
        pltpu.sync_copy(sig_hbm.at[b], sig_v)
        pltpu.sync_copy(thr_hbm.at[b], thr_v)
        t = thr_v[...]
        ones = jnp.full((LANES,), 1, jnp.int32)
        zeros = jnp.full((LANES,), 0, jnp.int32)
        cnt_v[...] = zeros

        def emit(pred):
            def step(j, _):
                v = sig_v[pl.ds(j * LANES, LANES)]
                m = pred(v)
                mi = jnp.where(m, ones, zeros)
                cnt_vec = cnt_v[...]
                pos = cnt_vec + plsc.cumsum(mi) - mi
                gidx = b * S + j * LANES + lax.iota(jnp.int32, LANES)
                plsc.store_scatter(idx_v, [pos], gidx, mask=m)
                plsc.store_scatter(wbuf_v, [pos], v, mask=m)
                cnt_v[...] = cnt_vec + plsc.all_reduce_population_count(m)
                return 0
            return step

        lax.fori_loop(0, S // LANES, emit(lambda v: v > t), 0)
        lax.fori_loop(0, S // LANES, emit(lambda v: v == t), 0)

        pltpu.sync_copy(wbuf_v.at[pl.ds(0, K)], w_hbm.at[b])
        pltpu.sync_copy(idx_v.at[pl.ds(0, K)], idx_hbm.at[pl.ds(b * K, K)])


def _emit(sig, thr):
    f = pl.kernel(
        _emit_body,
        out_type=[jax.ShapeDtypeStruct((BK,), jnp.int32),
                  jax.ShapeDtypeStruct((B, K), jnp.float32)],
        mesh=plsc.VectorSubcoreMesh(**_SC_MESH),
        scratch_types=[pltpu.VMEM((S,), jnp.float32),
                       pltpu.VMEM((S + 2 * LANES,), jnp.int32),
                       pltpu.VMEM((S + 2 * LANES,), jnp.float32),
                       pltpu.VMEM((LANES,), jnp.float32),
                       pltpu.VMEM((LANES,), jnp.int32)],
        compiler_params=pltpu.CompilerParams(needs_layout_passes=False),
    )
    return f(sig, thr)


def _gather_body(x_hbm, idx_hbm, xs_hbm, idx_v, rows_v, sem):
    wid = lax.axis_index("s") * NC + lax.axis_index("c")
    base = wid * ROWS_PER_W
    for c in range(NCHUNK):
        pltpu.sync_copy(idx_hbm.at[pl.ds(base + c * CHUNK, CHUNK)],
                        idx_v.at[c])
        pltpu.async_copy(x_hbm.at[idx_v.at[c]], rows_v, sem).wait()
        pltpu.sync_copy(rows_v, xs_hbm.at[pl.ds(base + c * CHUNK, CHUNK)])


def _gather(x_flat, idx):
    f = pl.kernel(
        _gather_body,
        out_type=jax.ShapeDtypeStruct((BK, D), jnp.float32),
        mesh=plsc.VectorSubcoreMesh(**_SC_MESH),
        scratch_types=[pltpu.VMEM((NCHUNK, CHUNK), jnp.int32),
                       pltpu.VMEM((CHUNK, D), jnp.float32),
                       pltpu.SemaphoreType.DMA],
    )
    return f(x_flat, idx)


FBLK = 512
NF = DFF // FBLK


def _ffn_body(xs_ref, w_ref, w1_ref, w2_ref, y_ref):
    f = pl.program_id(0)

    @pl.when(f == 0)
    def _():
        y_ref[...] = xs_ref[...].astype(jnp.float32)

    w1b = w1_ref[...].astype(jnp.bfloat16)
    h = jnp.maximum(
        jnp.dot(xs_ref[...], w1b, preferred_element_type=jnp.float32), 0.0)
    hb = h.astype(jnp.bfloat16)
    w2b = w2_ref[...].astype(jnp.bfloat16)
    y_ref[...] += w_ref[...] * jnp.dot(hb, w2b,
                                       preferred_element_type=jnp.float32)


def _ffn(xs_bf, w16, W1, W2):
    return pl.pallas_call(
        _ffn_body,
        grid=(NF,),
        in_specs=[
            pl.BlockSpec((BK, D), lambda f: (0, 0)),
            pl.BlockSpec((BK, 1), lambda f: (0, 0)),
            pl.BlockSpec((D, FBLK), lambda f: (0, f)),
            pl.BlockSpec((FBLK, D), lambda f: (f, 0)),
        ],
        out_specs=pl.BlockSpec((BK, D), lambda f: (0, 0)),
        out_shape=jax.ShapeDtypeStruct((BK, D), jnp.float32),
        compiler_params=pltpu.CompilerParams(
            dimension_semantics=("arbitrary",)),
    )(xs_bf, w16, W1, W2)


def _scatter_body(y_hbm, idx_hbm, out_ref, idx_v, rows_v, sem):
    wid = lax.axis_index("s") * NC + lax.axis_index("c")
    base = wid * ROWS_PER_W
    for c in range(NCHUNK):
        pltpu.sync_copy(idx_hbm.at[pl.ds(base + c * CHUNK, CHUNK)],
                        idx_v.at[c])
        pltpu.sync_copy(y_hbm.at[pl.ds(base + c * CHUNK, CHUNK)], rows_v)
        pltpu.async_copy(rows_v, out_ref.at[idx_v.at[c]], sem).wait()


def _scatter(y, idx, out_ref):
    f = pl.kernel(
        _scatter_body,
        out_type=(),
        mesh=plsc.VectorSubcoreMesh(**_SC_MESH),
        scratch_types=[pltpu.VMEM((NCHUNK, CHUNK), jnp.int32),
                       pltpu.VMEM((CHUNK, D), jnp.float32),
                       pltpu.SemaphoreType.DMA],
    )
    f(y, idx, out_ref)


def kernel(x, gate_W, W1, W2):
    sig = jax.nn.sigmoid(
        jnp.einsum('bsd,do->bso', x.astype(jnp.float32), gate_W))[..., 0]

    x_flat = x.reshape(B * S, D)
    thr = _thresholds(sig)
    idx, w = _emit(sig, thr)
    xs = _gather(x_flat, idx)
    out_copy = _copy(x)
    xs_bf = xs.astype(jnp.bfloat16)
    w16 = w.astype(jnp.float16).astype(jnp.float32).reshape(BK, 1)
    y = _ffn(xs_bf, w16, W1, W2)

    out_ref = jax.new_ref(out_copy.reshape(B * S, D))
    _scatter(y, idx, out_ref)
    return out_ref[...].reshape(B, S, D)
